# Initial kernel scaffold; baseline (speedup 1.0000x reference)
import jax
import jax.numpy as jnp
from jax.experimental import pallas as pl


def kernel(indices, codebook):
    raise NotImplementedError("write your pallas kernel here")


# speedup vs baseline: 1.5073x; 1.5073x over previous
import jax
import jax.numpy as jnp
from jax import lax
from jax.experimental import pallas as pl
from jax.experimental.pallas import tpu as pltpu
from jax.experimental.pallas import tpu_sc as plsc

N_CLASSES = 256
BINS = 8
LANES = 16

NC = 2
NS = 16
NW = NC * NS

B_DIM, L_DIM = 16384, 200
N = B_DIM * L_DIM
PER_W = N // NW
CHUNK = 6400
NCHUNK = PER_W // CHUNK
GROUPS = CHUNK // LANES


def _sc_body(idx_hbm, cb_hbm, out_hbm, cb_v, idx_v, out_v, sem_in, sem_out):
  wid = lax.axis_index("s") * NC + lax.axis_index("c")
  base = wid * PER_W

  pltpu.sync_copy(cb_hbm, cb_v)

  lane = lax.iota(jnp.int32, LANES)
  pat = lane * BINS

  def compute(buf):
    @plsc.parallel_loop(0, GROUPS, unroll=2)
    def _(k):
      iv = idx_v[buf][pl.ds(k * LANES, LANES)]
      cb_base = iv * BINS
      row = k * (LANES * BINS)
      for b in range(BINS):
        vals = plsc.load_gather(cb_v, [cb_base + b])
        plsc.store_scatter(out_v[buf], [pat + (row + b)], vals)

  cp_in = [None, None]
  cp_out = [None, None]

  def start_in(c, buf):
    cp_in[buf] = pltpu.async_copy(
        idx_hbm.at[pl.ds(base + c * CHUNK, CHUNK)], idx_v[buf], sem_in)

  def start_out(c, buf):
    cp_out[buf] = pltpu.async_copy(
        out_v[buf], out_hbm.at[pl.ds((base + c * CHUNK) * BINS, CHUNK * BINS)],
        sem_out)

  start_in(0, 0)
  for c in range(NCHUNK):
    buf = c % 2
    if c + 1 < NCHUNK:
      start_in(c + 1, 1 - buf)
    cp_in[buf].wait()
    if cp_out[buf] is not None:
      cp_out[buf].wait()
    compute(buf)
    start_out(c, buf)
  for buf in range(2):
    if cp_out[buf] is not None:
      cp_out[buf].wait()


@jax.jit
def kernel(indices, codebook):
  idx_flat = indices.reshape(N)
  cb_flat = codebook.reshape(N_CLASSES * BINS)
  mesh = plsc.VectorSubcoreMesh(
      core_axis_name="c", subcore_axis_name="s", num_cores=NC, num_subcores=NS)
  out = pl.kernel(
      _sc_body,
      out_type=jax.ShapeDtypeStruct((N * BINS,), jnp.float32),
      mesh=mesh,
      compiler_params=pltpu.CompilerParams(needs_layout_passes=False),
      scratch_types=[
          pltpu.VMEM((N_CLASSES * BINS,), jnp.float32),
          [pltpu.VMEM((CHUNK,), jnp.int32) for _ in range(2)],
          [pltpu.VMEM((CHUNK * BINS,), jnp.float32) for _ in range(2)],
          pltpu.SemaphoreType.DMA,
          pltpu.SemaphoreType.DMA,
      ],
  )(idx_flat, cb_flat)
  return out.reshape(B_DIM, L_DIM, 2, BINS // 2)


# speedup vs baseline: 131.4896x; 87.2368x over previous
import jax
import jax.numpy as jnp
from jax import lax
from jax.experimental import pallas as pl
from jax.experimental.pallas import tpu as pltpu
from jax.experimental.pallas import tpu_sc as plsc

N_CLASSES = 256
BINS = 8
LANES = 16

NC = 2
NS = 16
NW = NC * NS

B_DIM, L_DIM = 16384, 200
BT = B_DIM // 128
NB = 32
UNITS = L_DIM * (BT // NB)
UNITS_PER_W = UNITS // NW


def _sc_body(idx_hbm, cb_hbm, out_hbm, cb_v, ib, ob, sem_in, sem_out):
  wid = lax.axis_index("s") * NC + lax.axis_index("c")

  pltpu.sync_copy(cb_hbm, cb_v)

  def unit_coords(u):
    u_glob = wid * UNITS_PER_W + u
    l = lax.shift_right_logical(u_glob, 2)
    btc = lax.bitwise_and(u_glob, 3)
    return l, btc

  def compute(s):
    @plsc.parallel_loop(0, NB * 8, unroll=2)
    def _(t):
      row = lax.shift_right_logical(t, 3)
      col = lax.bitwise_and(t, 7) * LANES
      iv = ib[s][row, 0, pl.ds(col, LANES)]
      base = iv * BINS
      for c in range(2):
        for j in range(4):
          vals = plsc.load_gather(cb_v, [base + (4 * c + j)])
          ob[s][c][row, j, pl.ds(col, LANES)] = vals

  cp_in = [None, None]
  cp_out = [[None, None], [None, None]]

  def start_in(u, s):
    l, btc = unit_coords(u)
    lh = lax.shift_right_logical(l, 3)
    ll = lax.bitwise_and(l, 7)
    cp_in[s] = pltpu.async_copy(
        idx_hbm.at[lh, pl.ds(btc * NB, NB), pl.ds(ll, 1)],
        ib[s], sem_in)

  def start_out(u, s):
    l, btc = unit_coords(u)
    for c in range(2):
      cp_out[s][c] = pltpu.async_copy(
          ob[s][c], out_hbm.at[l, c, pl.ds(btc * NB, NB)], sem_out)

  start_in(0, 0)
  for u in range(UNITS_PER_W):
    s = u % 2
    if u + 1 < UNITS_PER_W:
      start_in(u + 1, 1 - s)
    cp_in[s].wait()
    if cp_out[s][0] is not None:
      cp_out[s][0].wait()
      cp_out[s][1].wait()
    compute(s)
    start_out(u, s)
  for s in range(2):
    if cp_out[s][0] is not None:
      cp_out[s][0].wait()
      cp_out[s][1].wait()


@jax.jit
def kernel(indices, codebook):
  idx_phys = indices.reshape(128, 128, 25, 8).transpose(2, 0, 3, 1)
  cb_flat = codebook.reshape(N_CLASSES * BINS)
  mesh = plsc.VectorSubcoreMesh(
      core_axis_name="c", subcore_axis_name="s", num_cores=NC, num_subcores=NS)
  out_phys = pl.kernel(
      _sc_body,
      out_type=jax.ShapeDtypeStruct((L_DIM, 2, BT, 4, 128), jnp.float32),
      mesh=mesh,
      compiler_params=pltpu.CompilerParams(needs_layout_passes=False),
      scratch_types=[
          pltpu.VMEM((N_CLASSES * BINS,), jnp.float32),
          [pltpu.VMEM((NB, 1, 128), jnp.int32) for _ in range(2)],
          [[pltpu.VMEM((NB, 4, 128), jnp.float32) for _ in range(2)]
           for _ in range(2)],
          pltpu.SemaphoreType.DMA,
          pltpu.SemaphoreType.DMA,
      ],
  )(idx_phys, cb_flat)
  return (out_phys.transpose(2, 4, 0, 1, 3).reshape(B_DIM, L_DIM, 2, 4))


# speedup vs baseline: 139.8910x; 1.0639x over previous
import jax
import jax.numpy as jnp
from jax import lax
from jax.experimental import pallas as pl
from jax.experimental.pallas import tpu as pltpu
from jax.experimental.pallas import tpu_sc as plsc

N_CLASSES = 256
BINS = 8
LANES = 16

NC = 2
NS = 16
NW = NC * NS

B_DIM, L_DIM = 16384, 200
BT = B_DIM // 128
NB = 32
UNITS = L_DIM * (BT // NB)
UNITS_PER_W = UNITS // NW


def _sc_body(idx_hbm, cb_hbm, out_hbm, cb_v, ib, ob, sem_in, sem_out):
  wid = lax.axis_index("s") * NC + lax.axis_index("c")

  pltpu.sync_copy(cb_hbm, cb_v)

  def unit_coords(u):
    u_glob = wid * UNITS_PER_W + u
    l = lax.shift_right_logical(u_glob, 2)
    btc = lax.bitwise_and(u_glob, 3)
    return l, btc

  def compute(s):
    @plsc.parallel_loop(0, NB * 8, unroll=2)
    def _(t):
      row = lax.shift_right_logical(t, 3)
      col = lax.bitwise_and(t, 7) * LANES
      iv = ib[s][row, 0, pl.ds(col, LANES)]
      base = iv * BINS
      for c in range(2):
        for j in range(4):
          vals = plsc.load_gather(cb_v, [base + (4 * c + j)])
          ob[s][c][row, j, pl.ds(col, LANES)] = vals

  def in_slice(u):
    l, btc = unit_coords(u)
    lh = lax.shift_right_logical(l, 3)
    ll = lax.bitwise_and(l, 7)
    return idx_hbm.at[lh, pl.ds(btc * NB, NB), pl.ds(ll, 1)]

  def start_in(u, s):
    pltpu.async_copy(in_slice(u), ib[s], sem_in)

  def wait_in(s):
    pltpu.make_async_copy(in_slice(0), ib[s], sem_in).wait()

  def start_out(u, s):
    l, btc = unit_coords(u)
    for c in range(2):
      pltpu.async_copy(ob[s][c], out_hbm.at[l, c, pl.ds(btc * NB, NB)],
                       sem_out)

  def wait_out(s):
    for c in range(2):
      pltpu.make_async_copy(ob[s][c], out_hbm.at[0, c, pl.ds(0, NB)],
                            sem_out).wait()

  def unit(u, s, first):
    wait_in(s)
    if not first:
      wait_out(s)
    compute(s)
    start_out(u, s)

  start_in(0, 0)
  start_in(1, 1)
  unit(0, 0, True)
  start_in(2, 0)
  unit(1, 1, True)
  start_in(3, 1)

  def body(i, _):
    for k in range(2):
      u = 2 * i + k
      unit(u, k, False)
      start_in(u + 2, k)
    return 0

  lax.fori_loop(1, 11, body, 0)

  unit(22, 0, False)
  start_in(24, 0)
  unit(23, 1, False)
  unit(24, 0, False)
  wait_out(1)
  wait_out(0)


@jax.jit
def kernel(indices, codebook):
  idx_phys = indices.reshape(128, 128, 25, 8).transpose(2, 0, 3, 1)
  cb_flat = codebook.reshape(N_CLASSES * BINS)
  mesh = plsc.VectorSubcoreMesh(
      core_axis_name="c", subcore_axis_name="s", num_cores=NC, num_subcores=NS)
  out_phys = pl.kernel(
      _sc_body,
      out_type=jax.ShapeDtypeStruct((L_DIM, 2, BT, 4, 128), jnp.float32),
      mesh=mesh,
      compiler_params=pltpu.CompilerParams(needs_layout_passes=False),
      scratch_types=[
          pltpu.VMEM((N_CLASSES * BINS,), jnp.float32),
          [pltpu.VMEM((NB, 1, 128), jnp.int32) for _ in range(2)],
          [[pltpu.VMEM((NB, 4, 128), jnp.float32) for _ in range(2)]
           for _ in range(2)],
          pltpu.SemaphoreType.DMA,
          pltpu.SemaphoreType.DMA,
      ],
  )(idx_phys, cb_flat)
  return (out_phys.transpose(2, 4, 0, 1, 3).reshape(B_DIM, L_DIM, 2, 4))
